# baseline (speedup 1.0000x reference)
import functools

import jax
import jax.numpy as jnp
from jax import lax
from jax.experimental import pallas as pl
from jax.experimental.pallas import tpu as pltpu
from jax.experimental.pallas import tpu_sc as plsc

HIDDEN = 64
BATCH = 16384
NUM_CLASSES = 10

NC = 2
NS = 16
NW = NC * NS
BPW = BATCH // NW
CHUNK = 128
NCHUNK = BPW // CHUNK


def _sc_gather_body(idx_hbm, tpt_hbm, trt_hbm, hpt_hbm,
                    tp_out, tr_out, hp_out,
                    idx_v, rows_tp, rows_tr, rows_hp, sem):
    wid = lax.axis_index("s") * NC + lax.axis_index("c")
    base = wid * BPW
    pltpu.sync_copy(idx_hbm.at[wid], idx_v)
    descs = []
    for t, (tab, rows) in enumerate(
            ((tpt_hbm, rows_tp), (trt_hbm, rows_tr), (hpt_hbm, rows_hp))):
        for j in range(NCHUNK):
            descs.append(pltpu.async_copy(
                tab.at[idx_v.at[t, j]],
                rows.at[pl.ds(j * CHUNK, CHUNK)],
                sem))
    for d in descs:
        d.wait()
    pltpu.sync_copy(rows_tp, tp_out.at[pl.ds(base, BPW)])
    pltpu.sync_copy(rows_tr, tr_out.at[pl.ds(base, BPW)])
    pltpu.sync_copy(rows_hp, hp_out.at[pl.ds(base, BPW)])


@functools.cache
def _sc_gather():
    return functools.partial(
        pl.kernel,
        out_type=[jax.ShapeDtypeStruct((BATCH, HIDDEN), jnp.float32)] * 3,
        mesh=plsc.VectorSubcoreMesh(
            core_axis_name="c", subcore_axis_name="s",
            num_cores=NC, num_subcores=NS),
        scratch_types=[
            pltpu.VMEM((3, NCHUNK, CHUNK), jnp.int32),
            pltpu.VMEM((BPW, HIDDEN), jnp.float32),
            pltpu.VMEM((BPW, HIDDEN), jnp.float32),
            pltpu.VMEM((BPW, HIDDEN), jnp.float32),
            pltpu.SemaphoreType.DMA,
        ],
        compiler_params=pltpu.CompilerParams(use_tc_tiling_on_sc=False),
    )(_sc_gather_body)


def _tc_dense_body(tp_ref, tr_ref, hp_ref, w_ref, b_ref, o_ref):
    logits = (
        jnp.dot(tp_ref[...], w_ref[0:HIDDEN],
                preferred_element_type=jnp.float32)
        + jnp.dot(tr_ref[...], w_ref[HIDDEN:2 * HIDDEN],
                  preferred_element_type=jnp.float32)
        + jnp.dot(hp_ref[...], w_ref[2 * HIDDEN:3 * HIDDEN],
                  preferred_element_type=jnp.float32)
        + b_ref[...]
    )
    m = jnp.max(logits, axis=-1, keepdims=True)
    e = jnp.exp(logits - m)
    o_ref[...] = e / jnp.sum(e, axis=-1, keepdims=True)


def _tc_dense(tp, tr, hp, w, b2d, block):
    grid = (BATCH // block,)
    act_spec = pl.BlockSpec((block, HIDDEN), lambda i: (i, 0))
    return pl.pallas_call(
        _tc_dense_body,
        grid=grid,
        in_specs=[
            act_spec, act_spec, act_spec,
            pl.BlockSpec((3 * HIDDEN, NUM_CLASSES), lambda i: (0, 0)),
            pl.BlockSpec((1, NUM_CLASSES), lambda i: (0, 0)),
        ],
        out_specs=pl.BlockSpec((block, NUM_CLASSES), lambda i: (i, 0)),
        out_shape=jax.ShapeDtypeStruct((BATCH, NUM_CLASSES), jnp.float32),
    )(tp, tr, hp, w, b2d)


def kernel(x, target_pos_table, target_rot_table, hand_pos_table, W, b):
    idx = (x.astype(jnp.int32)
           .reshape(NW, NCHUNK, CHUNK, 3)
           .transpose(0, 3, 1, 2))
    tp, tr, hp = _sc_gather()(
        idx, target_pos_table, target_rot_table, hand_pos_table)
    return _tc_dense(tp, tr, hp, W, b.reshape(1, NUM_CLASSES), block=2048)
